# baseline (speedup 1.0000x reference)
import functools

import jax
import jax.numpy as jnp
from jax import lax
from jax.experimental import pallas as pl
from jax.experimental.pallas import tpu as pltpu
from jax.experimental.pallas import tpu_sc as plsc

B = 128
S = 80
VOCAB = 10000
EMB = 100
EMB_PAD = 128
UNITS = 64
NTOK = B * S

NC = 2
NS = 16
NW = NC * NS
ROWS_PER_W = NTOK // NW


@functools.lru_cache(maxsize=1)
def _make_sc_gather():
    mesh = plsc.VectorSubcoreMesh(core_axis_name="c", subcore_axis_name="s")

    @functools.partial(
        pl.kernel,
        mesh=mesh,
        out_type=jax.ShapeDtypeStruct((NTOK, EMB_PAD), jnp.float32),
        scratch_types=[
            pltpu.VMEM((ROWS_PER_W,), jnp.int32),
            pltpu.VMEM((ROWS_PER_W, EMB_PAD), jnp.float32),
            pltpu.SemaphoreType.DMA,
        ],
    )
    def _sc_gather(table_hbm, idx_hbm, out_hbm, idx_v, rows_v, sem):
        wid = lax.axis_index("s") * NC + lax.axis_index("c")
        base = wid * ROWS_PER_W
        pltpu.sync_copy(idx_hbm.at[pl.ds(base, ROWS_PER_W)], idx_v)
        pltpu.async_copy(table_hbm.at[idx_v], rows_v, sem).wait()
        pltpu.sync_copy(rows_v, out_hbm.at[pl.ds(base, ROWS_PER_W)])

    return _sc_gather


def _rnn_body(xs_ref, w_ref, bias_ref, wd1_ref, bd1_ref, wd2_ref,
              bd2_ref, out_ref):
    H2 = 2 * UNITS

    def gates(Z):
        T = jnp.tanh(Z)
        U = 0.5 * T + 0.5
        return U[:, 0:H2], U[:, H2:2 * H2], T[:, 2 * H2:3 * H2], \
            U[:, 3 * H2:4 * H2]

    colmask = lax.broadcasted_iota(jnp.int32, (B, H2), 1) >= UNITS
    Z0 = jnp.dot(xs_ref[pl.ds(0, B), :], w_ref[pl.ds(H2, EMB_PAD), :],
                 preferred_element_type=jnp.float32) + bias_ref[...]
    i, f, g, o = gates(Z0)
    C = jnp.where(colmask, 0.0, i * g)
    H = jnp.where(colmask, 0.0, o * jnp.tanh(C))

    def step(r, carry):
        H, C = carry
        tx = jnp.minimum(r, S - 1) * B
        A = jnp.concatenate([H, xs_ref[pl.ds(tx, B), :]],
                            axis=1).astype(jnp.bfloat16)
        Z = jnp.dot(A, w_ref[...].astype(jnp.bfloat16),
                    preferred_element_type=jnp.float32) + bias_ref[...]
        i, f, g, o = gates(Z)
        C = f * C + i * g
        H = o * jnp.tanh(C)
        return H, C

    H, C = lax.fori_loop(1, S + 1, step, (H, C))
    h1 = H[:, UNITS:H2]

    hidden = jnp.maximum(
        jnp.dot(h1, wd1_ref[...], preferred_element_type=jnp.float32)
        + bd1_ref[...], 0.0)
    logits = jnp.dot(hidden, wd2_ref[...],
                     preferred_element_type=jnp.float32) + bd2_ref[...]
    out_ref[...] = jax.nn.sigmoid(logits)


def _build_weights(Wk0, Wr0, b0, Wk1, Wr1, b1):
    wk0_pad = jnp.concatenate(
        [Wk0, jnp.zeros((EMB_PAD - EMB, 4 * UNITS), Wk0.dtype)], axis=0)
    z64 = jnp.zeros((UNITS, UNITS), jnp.float32)
    z128x = jnp.zeros((EMB_PAD, UNITS), jnp.float32)
    cols = []
    bias = []
    for gi, gate in enumerate("ifgo"):
        s = 1.0 if gate == "g" else 0.5
        sl = slice(gi * UNITS, (gi + 1) * UNITS)
        cols.append(s * jnp.concatenate(
            [Wr0[:, sl], z64, wk0_pad[:, sl]], axis=0))
        cols.append(jnp.concatenate(
            [s * Wk1[:, sl], s * Wr1[:, sl], z128x], axis=0))
        bias.append(s * b0[sl])
        bias.append(s * b1[sl])
    w_big = jnp.concatenate(cols, axis=1)
    bias_big = jnp.concatenate(bias).reshape(1, 8 * UNITS)
    return w_big, bias_big


def _rnn_call(xs, w_big, bias_big, Wd1, bd1, Wd2, bd2, interpret=False):
    return pl.pallas_call(
        _rnn_body,
        out_shape=jax.ShapeDtypeStruct((B, 1), jnp.float32),
        interpret=interpret,
    )(xs, w_big, bias_big, Wd1, bd1.reshape(1, UNITS), Wd2,
      bd2.reshape(1, 1))


def kernel(inputs, emb, Wk0, Wr0, b0, Wk1, Wr1, b1, Wd1, bd1, Wd2, bd2):
    emb_pad = jnp.concatenate(
        [emb, jnp.zeros((VOCAB, EMB_PAD - EMB), emb.dtype)], axis=1)
    idx = jnp.transpose(inputs).reshape(NTOK)
    xs = _make_sc_gather()(emb_pad, idx)
    w_big, bias_big = _build_weights(Wk0, Wr0, b0, Wk1, Wr1, b1)
    return _rnn_call(xs, w_big, bias_big, Wd1, bd1, Wd2, bd2)


# speedup vs baseline: 1.0017x; 1.0017x over previous
import functools

import jax
import jax.numpy as jnp
from jax import lax
from jax.experimental import pallas as pl
from jax.experimental.pallas import tpu as pltpu
from jax.experimental.pallas import tpu_sc as plsc

B = 128
S = 80
VOCAB = 10000
EMB = 100
EMB_PAD = 128
UNITS = 64
NTOK = B * S

NC = 2
NS = 16
NW = NC * NS
ROWS_PER_W = NTOK // NW


@functools.lru_cache(maxsize=1)
def _make_sc_gather():
    mesh = plsc.VectorSubcoreMesh(core_axis_name="c", subcore_axis_name="s")

    @functools.partial(
        pl.kernel,
        mesh=mesh,
        out_type=jax.ShapeDtypeStruct((NTOK, EMB_PAD), jnp.float32),
        scratch_types=[
            pltpu.VMEM((ROWS_PER_W,), jnp.int32),
            pltpu.VMEM((ROWS_PER_W, EMB_PAD), jnp.float32),
            pltpu.SemaphoreType.DMA,
        ],
    )
    def _sc_gather(table_hbm, idx_hbm, out_hbm, idx_v, rows_v, sem):
        wid = lax.axis_index("s") * NC + lax.axis_index("c")
        base = wid * ROWS_PER_W
        pltpu.sync_copy(idx_hbm.at[pl.ds(base, ROWS_PER_W)], idx_v)
        pltpu.async_copy(table_hbm.at[idx_v], rows_v, sem).wait()
        pltpu.sync_copy(rows_v, out_hbm.at[pl.ds(base, ROWS_PER_W)])

    return _sc_gather


def _rnn_body(xs_ref, w_ref, bias_ref, wd1_ref, bd1_ref, wd2_ref,
              bd2_ref, out_ref):
    H2 = 2 * UNITS

    def gates(Z):
        T = jnp.tanh(Z)
        U = 0.5 * T + 0.5
        return U[:, 0:H2], U[:, H2:2 * H2], T[:, 2 * H2:3 * H2], \
            U[:, 3 * H2:4 * H2]

    colmask = lax.broadcasted_iota(jnp.int32, (B, H2), 1) >= UNITS
    Z0 = jnp.dot(xs_ref[pl.ds(0, B), :], w_ref[pl.ds(H2, EMB_PAD), :],
                 preferred_element_type=jnp.float32) + bias_ref[...]
    i, f, g, o = gates(Z0)
    C = jnp.where(colmask, 0.0, i * g)
    H = jnp.where(colmask, 0.0, o * jnp.tanh(C))

    HB = B // 2

    def step(r, carry):
        Ha, Ca, Hb, Cb = carry
        tx = jnp.minimum(r, S - 1) * B
        Aa = jnp.concatenate([Ha, xs_ref[pl.ds(tx, HB), :]], axis=1)
        Ab = jnp.concatenate([Hb, xs_ref[pl.ds(tx + HB, HB), :]], axis=1)
        Za = jnp.dot(Aa, w_ref[...],
                     preferred_element_type=jnp.float32) + bias_ref[...]
        Zb = jnp.dot(Ab, w_ref[...],
                     preferred_element_type=jnp.float32) + bias_ref[...]
        ia, fa, ga, oa = gates(Za)
        ib, fb, gb, ob = gates(Zb)
        Ca = fa * Ca + ia * ga
        Cb = fb * Cb + ib * gb
        Ha = oa * jnp.tanh(Ca)
        Hb = ob * jnp.tanh(Cb)
        return Ha, Ca, Hb, Cb

    Ha, Ca, Hb, Cb = lax.fori_loop(
        1, S + 1, step,
        (H[0:HB], C[0:HB], H[HB:B], C[HB:B]))
    H = jnp.concatenate([Ha, Hb], axis=0)
    h1 = H[:, UNITS:H2]

    hidden = jnp.maximum(
        jnp.dot(h1, wd1_ref[...], preferred_element_type=jnp.float32)
        + bd1_ref[...], 0.0)
    logits = jnp.dot(hidden, wd2_ref[...],
                     preferred_element_type=jnp.float32) + bd2_ref[...]
    out_ref[...] = jax.nn.sigmoid(logits)


def _build_weights(Wk0, Wr0, b0, Wk1, Wr1, b1):
    wk0_pad = jnp.concatenate(
        [Wk0, jnp.zeros((EMB_PAD - EMB, 4 * UNITS), Wk0.dtype)], axis=0)
    z64 = jnp.zeros((UNITS, UNITS), jnp.float32)
    z128x = jnp.zeros((EMB_PAD, UNITS), jnp.float32)
    cols = []
    bias = []
    for gi, gate in enumerate("ifgo"):
        s = 1.0 if gate == "g" else 0.5
        sl = slice(gi * UNITS, (gi + 1) * UNITS)
        cols.append(s * jnp.concatenate(
            [Wr0[:, sl], z64, wk0_pad[:, sl]], axis=0))
        cols.append(jnp.concatenate(
            [s * Wk1[:, sl], s * Wr1[:, sl], z128x], axis=0))
        bias.append(s * b0[sl])
        bias.append(s * b1[sl])
    w_big = jnp.concatenate(cols, axis=1)
    bias_big = jnp.concatenate(bias).reshape(1, 8 * UNITS)
    return w_big, bias_big


def _rnn_call(xs, w_big, bias_big, Wd1, bd1, Wd2, bd2, interpret=False):
    return pl.pallas_call(
        _rnn_body,
        out_shape=jax.ShapeDtypeStruct((B, 1), jnp.float32),
        interpret=interpret,
    )(xs, w_big, bias_big, Wd1, bd1.reshape(1, UNITS), Wd2,
      bd2.reshape(1, 1))


def kernel(inputs, emb, Wk0, Wr0, b0, Wk1, Wr1, b1, Wd1, bd1, Wd2, bd2):
    emb_pad = jnp.concatenate(
        [emb, jnp.zeros((VOCAB, EMB_PAD - EMB), emb.dtype)], axis=1)
    idx = jnp.transpose(inputs).reshape(NTOK)
    xs = _make_sc_gather()(emb_pad, idx)
    w_big, bias_big = _build_weights(Wk0, Wr0, b0, Wk1, Wr1, b1)
    return _rnn_call(xs, w_big, bias_big, Wd1, bd1, Wd2, bd2)


# speedup vs baseline: 1.0108x; 1.0091x over previous
import functools

import jax
import jax.numpy as jnp
from jax import lax
from jax.experimental import pallas as pl
from jax.experimental.pallas import tpu as pltpu
from jax.experimental.pallas import tpu_sc as plsc

B = 128
S = 80
VOCAB = 10000
EMB = 100
EMB_PAD = 128
UNITS = 64
NTOK = B * S

NC = 2
NS = 16
NW = NC * NS
ROWS_PER_W = NTOK // NW


@functools.lru_cache(maxsize=1)
def _make_sc_gather():
    mesh = plsc.VectorSubcoreMesh(core_axis_name="c", subcore_axis_name="s")

    @functools.partial(
        pl.kernel,
        mesh=mesh,
        out_type=jax.ShapeDtypeStruct((NTOK, EMB_PAD), jnp.float32),
        scratch_types=[
            pltpu.VMEM((ROWS_PER_W,), jnp.int32),
            pltpu.VMEM((ROWS_PER_W, EMB_PAD), jnp.float32),
            pltpu.SemaphoreType.DMA,
        ],
    )
    def _sc_gather(table_hbm, idx_hbm, out_hbm, idx_v, rows_v, sem):
        wid = lax.axis_index("s") * NC + lax.axis_index("c")
        base = wid * ROWS_PER_W
        pltpu.sync_copy(idx_hbm.at[pl.ds(base, ROWS_PER_W)], idx_v)
        pltpu.async_copy(table_hbm.at[idx_v], rows_v, sem).wait()
        pltpu.sync_copy(rows_v, out_hbm.at[pl.ds(base, ROWS_PER_W)])

    return _sc_gather


def _rnn_body(xs_ref, w_ref, bias_ref, wd1_ref, bd1_ref, wd2_ref,
              bd2_ref, out_ref):
    H2 = 2 * UNITS

    def gates(Z):
        T = jnp.tanh(Z)
        U = 0.5 * T + 0.5
        return U[:, 0:H2], U[:, H2:2 * H2], T[:, 2 * H2:3 * H2], \
            U[:, 3 * H2:4 * H2]

    colmask = lax.broadcasted_iota(jnp.int32, (B, H2), 1) >= UNITS
    Z0 = jnp.dot(xs_ref[pl.ds(0, B), :], w_ref[pl.ds(H2, EMB_PAD), :],
                 preferred_element_type=jnp.float32) + bias_ref[...]
    i, f, g, o = gates(Z0)
    C = jnp.where(colmask, 0.0, i * g)
    H = jnp.where(colmask, 0.0, o * jnp.tanh(C))

    HB = B // 2
    Ha, Ca, Hb, Cb = H[0:HB], C[0:HB], H[HB:B], C[HB:B]
    Zb = jnp.dot(jnp.concatenate([Hb, xs_ref[pl.ds(B + HB, HB), :]], axis=1),
                 w_ref[...], preferred_element_type=jnp.float32) + bias_ref[...]

    def step(r, carry):
        Ha, Ca, Hb, Cb, Zb = carry
        ib, fb, gb, ob = gates(Zb)
        Cb = fb * Cb + ib * gb
        Hb = ob * jnp.tanh(Cb)
        tx = jnp.minimum(r, S - 1) * B
        Aa = jnp.concatenate([Ha, xs_ref[pl.ds(tx, HB), :]], axis=1)
        Za = jnp.dot(Aa, w_ref[...],
                     preferred_element_type=jnp.float32) + bias_ref[...]
        ia, fa, ga, oa = gates(Za)
        Ca = fa * Ca + ia * ga
        Ha = oa * jnp.tanh(Ca)
        tx2 = jnp.minimum(r + 1, S - 1) * B
        Ab = jnp.concatenate([Hb, xs_ref[pl.ds(tx2 + HB, HB), :]], axis=1)
        Zb = jnp.dot(Ab, w_ref[...],
                     preferred_element_type=jnp.float32) + bias_ref[...]
        return Ha, Ca, Hb, Cb, Zb

    Ha, Ca, Hb, Cb, Zb = lax.fori_loop(1, S + 1, step, (Ha, Ca, Hb, Cb, Zb))
    H = jnp.concatenate([Ha, Hb], axis=0)
    h1 = H[:, UNITS:H2]

    hidden = jnp.maximum(
        jnp.dot(h1, wd1_ref[...], preferred_element_type=jnp.float32)
        + bd1_ref[...], 0.0)
    logits = jnp.dot(hidden, wd2_ref[...],
                     preferred_element_type=jnp.float32) + bd2_ref[...]
    out_ref[...] = jax.nn.sigmoid(logits)


def _build_weights(Wk0, Wr0, b0, Wk1, Wr1, b1):
    wk0_pad = jnp.concatenate(
        [Wk0, jnp.zeros((EMB_PAD - EMB, 4 * UNITS), Wk0.dtype)], axis=0)
    z64 = jnp.zeros((UNITS, UNITS), jnp.float32)
    z128x = jnp.zeros((EMB_PAD, UNITS), jnp.float32)
    cols = []
    bias = []
    for gi, gate in enumerate("ifgo"):
        s = 1.0 if gate == "g" else 0.5
        sl = slice(gi * UNITS, (gi + 1) * UNITS)
        cols.append(s * jnp.concatenate(
            [Wr0[:, sl], z64, wk0_pad[:, sl]], axis=0))
        cols.append(jnp.concatenate(
            [s * Wk1[:, sl], s * Wr1[:, sl], z128x], axis=0))
        bias.append(s * b0[sl])
        bias.append(s * b1[sl])
    w_big = jnp.concatenate(cols, axis=1)
    bias_big = jnp.concatenate(bias).reshape(1, 8 * UNITS)
    return w_big, bias_big


def _rnn_call(xs, w_big, bias_big, Wd1, bd1, Wd2, bd2, interpret=False):
    return pl.pallas_call(
        _rnn_body,
        out_shape=jax.ShapeDtypeStruct((B, 1), jnp.float32),
        interpret=interpret,
    )(xs, w_big, bias_big, Wd1, bd1.reshape(1, UNITS), Wd2,
      bd2.reshape(1, 1))


def kernel(inputs, emb, Wk0, Wr0, b0, Wk1, Wr1, b1, Wd1, bd1, Wd2, bd2):
    emb_pad = jnp.concatenate(
        [emb, jnp.zeros((VOCAB, EMB_PAD - EMB), emb.dtype)], axis=1)
    idx = jnp.transpose(inputs).reshape(NTOK)
    xs = _make_sc_gather()(emb_pad, idx)
    w_big, bias_big = _build_weights(Wk0, Wr0, b0, Wk1, Wr1, b1)
    return _rnn_call(xs, w_big, bias_big, Wd1, bd1, Wd2, bd2)


# speedup vs baseline: 1.0592x; 1.0479x over previous
import functools

import jax
import jax.numpy as jnp
from jax import lax
from jax.experimental import pallas as pl
from jax.experimental.pallas import tpu as pltpu
from jax.experimental.pallas import tpu_sc as plsc

B = 128
S = 80
VOCAB = 10000
EMB = 100
EMB_PAD = 128
UNITS = 64
NTOK = B * S

NC = 2
NS = 16
NW = NC * NS
ROWS_PER_W = NTOK // NW


@functools.lru_cache(maxsize=1)
def _make_sc_gather():
    mesh = plsc.VectorSubcoreMesh(core_axis_name="c", subcore_axis_name="s")

    @functools.partial(
        pl.kernel,
        mesh=mesh,
        out_type=jax.ShapeDtypeStruct((NTOK, EMB_PAD), jnp.float32),
        scratch_types=[
            pltpu.VMEM((ROWS_PER_W,), jnp.int32),
            pltpu.VMEM((ROWS_PER_W, EMB_PAD), jnp.float32),
            pltpu.SemaphoreType.DMA,
        ],
    )
    def _sc_gather(table_hbm, idx_hbm, out_hbm, idx_v, rows_v, sem):
        wid = lax.axis_index("s") * NC + lax.axis_index("c")
        base = wid * ROWS_PER_W
        pltpu.sync_copy(idx_hbm.at[pl.ds(base, ROWS_PER_W)], idx_v)
        pltpu.async_copy(table_hbm.at[idx_v], rows_v, sem).wait()
        pltpu.sync_copy(rows_v, out_hbm.at[pl.ds(base, ROWS_PER_W)])

    return _sc_gather


def _rnn_body(xs_ref, w_ref, bias_ref, wd1_ref, bd1_ref, wd2_ref,
              bd2_ref, out_ref):
    H2 = 2 * UNITS

    def gates(Z):
        T = jnp.tanh(Z)
        U = 0.5 * T + 0.5
        return U[:, 0:H2], U[:, H2:2 * H2], T[:, 2 * H2:3 * H2], \
            U[:, 3 * H2:4 * H2]

    colmask = lax.broadcasted_iota(jnp.int32, (B, H2), 1) >= UNITS
    Z0 = jnp.dot(xs_ref[pl.ds(0, B), :], w_ref[pl.ds(H2, EMB_PAD), :],
                 preferred_element_type=jnp.float32) + bias_ref[...]
    i, f, g, o = gates(Z0)
    C = jnp.where(colmask, 0.0, i * g)
    H = jnp.where(colmask, 0.0, o * jnp.tanh(C))

    HB = B // 2
    Ha, Ca, Hb, Cb = H[0:HB], C[0:HB], H[HB:B], C[HB:B]
    Zb = jnp.dot(jnp.concatenate([Hb, xs_ref[pl.ds(B + HB, HB), :]], axis=1),
                 w_ref[...], preferred_element_type=jnp.float32) + bias_ref[...]

    def step(r, carry):
        Ha, Ca, Hb, Cb, Zb = carry
        ib, fb, gb, ob = gates(Zb)
        Cb = fb * Cb + ib * gb
        Hb = ob * jnp.tanh(Cb)
        tx = jnp.minimum(r, S - 1) * B
        Aa = jnp.concatenate([Ha, xs_ref[pl.ds(tx, HB), :]], axis=1)
        Za = jnp.dot(Aa, w_ref[...],
                     preferred_element_type=jnp.float32) + bias_ref[...]
        ia, fa, ga, oa = gates(Za)
        Ca = fa * Ca + ia * ga
        Ha = oa * jnp.tanh(Ca)
        tx2 = jnp.minimum(r + 1, S - 1) * B
        Ab = jnp.concatenate([Hb, xs_ref[pl.ds(tx2 + HB, HB), :]], axis=1)
        Zb = jnp.dot(Ab, w_ref[...],
                     preferred_element_type=jnp.float32) + bias_ref[...]
        return Ha, Ca, Hb, Cb, Zb

    Ha, Ca, Hb, Cb, Zb = lax.fori_loop(1, S + 1, step, (Ha, Ca, Hb, Cb, Zb),
                                       unroll=2)
    H = jnp.concatenate([Ha, Hb], axis=0)
    h1 = H[:, UNITS:H2]

    hidden = jnp.maximum(
        jnp.dot(h1, wd1_ref[...], preferred_element_type=jnp.float32)
        + bd1_ref[...], 0.0)
    logits = jnp.dot(hidden, wd2_ref[...],
                     preferred_element_type=jnp.float32) + bd2_ref[...]
    out_ref[...] = jax.nn.sigmoid(logits)


def _build_weights(Wk0, Wr0, b0, Wk1, Wr1, b1):
    wk0_pad = jnp.concatenate(
        [Wk0, jnp.zeros((EMB_PAD - EMB, 4 * UNITS), Wk0.dtype)], axis=0)
    z64 = jnp.zeros((UNITS, UNITS), jnp.float32)
    z128x = jnp.zeros((EMB_PAD, UNITS), jnp.float32)
    cols = []
    bias = []
    for gi, gate in enumerate("ifgo"):
        s = 1.0 if gate == "g" else 0.5
        sl = slice(gi * UNITS, (gi + 1) * UNITS)
        cols.append(s * jnp.concatenate(
            [Wr0[:, sl], z64, wk0_pad[:, sl]], axis=0))
        cols.append(jnp.concatenate(
            [s * Wk1[:, sl], s * Wr1[:, sl], z128x], axis=0))
        bias.append(s * b0[sl])
        bias.append(s * b1[sl])
    w_big = jnp.concatenate(cols, axis=1)
    bias_big = jnp.concatenate(bias).reshape(1, 8 * UNITS)
    return w_big, bias_big


def _rnn_call(xs, w_big, bias_big, Wd1, bd1, Wd2, bd2, interpret=False):
    return pl.pallas_call(
        _rnn_body,
        out_shape=jax.ShapeDtypeStruct((B, 1), jnp.float32),
        interpret=interpret,
    )(xs, w_big, bias_big, Wd1, bd1.reshape(1, UNITS), Wd2,
      bd2.reshape(1, 1))


def kernel(inputs, emb, Wk0, Wr0, b0, Wk1, Wr1, b1, Wd1, bd1, Wd2, bd2):
    emb_pad = jnp.concatenate(
        [emb, jnp.zeros((VOCAB, EMB_PAD - EMB), emb.dtype)], axis=1)
    idx = jnp.transpose(inputs).reshape(NTOK)
    xs = _make_sc_gather()(emb_pad, idx)
    w_big, bias_big = _build_weights(Wk0, Wr0, b0, Wk1, Wr1, b1)
    return _rnn_call(xs, w_big, bias_big, Wd1, bd1, Wd2, bd2)


# speedup vs baseline: 1.1098x; 1.0478x over previous
import functools

import jax
import jax.numpy as jnp
from jax import lax
from jax.experimental import pallas as pl
from jax.experimental.pallas import tpu as pltpu
from jax.experimental.pallas import tpu_sc as plsc

B = 128
S = 80
VOCAB = 10000
EMB = 100
EMB_PAD = 128
UNITS = 64
NTOK = B * S

NC = 2
NS = 16
NW = NC * NS
ROWS_PER_W = NTOK // NW


@functools.lru_cache(maxsize=1)
def _make_sc_gather():
    mesh = plsc.VectorSubcoreMesh(core_axis_name="c", subcore_axis_name="s")

    @functools.partial(
        pl.kernel,
        mesh=mesh,
        out_type=jax.ShapeDtypeStruct((NTOK, EMB_PAD), jnp.float32),
        scratch_types=[
            pltpu.VMEM((ROWS_PER_W,), jnp.int32),
            pltpu.VMEM((ROWS_PER_W, EMB_PAD), jnp.float32),
            pltpu.SemaphoreType.DMA,
        ],
    )
    def _sc_gather(table_hbm, idx_hbm, out_hbm, idx_v, rows_v, sem):
        wid = lax.axis_index("s") * NC + lax.axis_index("c")
        base = wid * ROWS_PER_W
        pltpu.sync_copy(idx_hbm.at[pl.ds(base, ROWS_PER_W)], idx_v)
        pltpu.async_copy(table_hbm.at[idx_v], rows_v, sem).wait()
        pltpu.sync_copy(rows_v, out_hbm.at[pl.ds(base, ROWS_PER_W)])

    return _sc_gather


def _rnn_body(xs_ref, w_ref, bias_ref, wd1_ref, bd1_ref, wd2_ref,
              bd2_ref, out_ref):
    H2 = 2 * UNITS

    def gates(Z):
        T = jnp.tanh(Z)
        U = 0.5 * T + 0.5
        return U[:, 0:H2], U[:, H2:2 * H2], T[:, 2 * H2:3 * H2], \
            U[:, 3 * H2:4 * H2]

    colmask = lax.broadcasted_iota(jnp.int32, (B, H2), 1) >= UNITS
    Z0 = jnp.dot(xs_ref[pl.ds(0, B), :], w_ref[pl.ds(H2, EMB_PAD), :],
                 preferred_element_type=jnp.float32) + bias_ref[...]
    i, f, g, o = gates(Z0)
    C = jnp.where(colmask, 0.0, i * g)
    H = jnp.where(colmask, 0.0, o * jnp.tanh(C))

    HB = B // 2
    Ha, Ca, Hb, Cb = H[0:HB], C[0:HB], H[HB:B], C[HB:B]
    Zb = jnp.dot(jnp.concatenate([Hb, xs_ref[pl.ds(B + HB, HB), :]], axis=1),
                 w_ref[...], preferred_element_type=jnp.float32) + bias_ref[...]

    def step(r, carry):
        Ha, Ca, Hb, Cb, Zb = carry
        ib, fb, gb, ob = gates(Zb)
        Cb = fb * Cb + ib * gb
        Hb = ob * jnp.tanh(Cb)
        tx = jnp.minimum(r, S - 1) * B
        Aa = jnp.concatenate([Ha, xs_ref[pl.ds(tx, HB), :]], axis=1)
        Za = jnp.dot(Aa, w_ref[...],
                     preferred_element_type=jnp.float32) + bias_ref[...]
        ia, fa, ga, oa = gates(Za)
        Ca = fa * Ca + ia * ga
        Ha = oa * jnp.tanh(Ca)
        tx2 = jnp.minimum(r + 1, S - 1) * B
        Ab = jnp.concatenate([Hb, xs_ref[pl.ds(tx2 + HB, HB), :]], axis=1)
        Zb = jnp.dot(Ab, w_ref[...],
                     preferred_element_type=jnp.float32) + bias_ref[...]
        return Ha, Ca, Hb, Cb, Zb

    Ha, Ca, Hb, Cb, Zb = lax.fori_loop(1, S + 1, step, (Ha, Ca, Hb, Cb, Zb),
                                       unroll=4)
    H = jnp.concatenate([Ha, Hb], axis=0)
    h1 = H[:, UNITS:H2]

    hidden = jnp.maximum(
        jnp.dot(h1, wd1_ref[...], preferred_element_type=jnp.float32)
        + bd1_ref[...], 0.0)
    logits = jnp.dot(hidden, wd2_ref[...],
                     preferred_element_type=jnp.float32) + bd2_ref[...]
    out_ref[...] = jax.nn.sigmoid(logits)


def _build_weights(Wk0, Wr0, b0, Wk1, Wr1, b1):
    wk0_pad = jnp.concatenate(
        [Wk0, jnp.zeros((EMB_PAD - EMB, 4 * UNITS), Wk0.dtype)], axis=0)
    z64 = jnp.zeros((UNITS, UNITS), jnp.float32)
    z128x = jnp.zeros((EMB_PAD, UNITS), jnp.float32)
    cols = []
    bias = []
    for gi, gate in enumerate("ifgo"):
        s = 1.0 if gate == "g" else 0.5
        sl = slice(gi * UNITS, (gi + 1) * UNITS)
        cols.append(s * jnp.concatenate(
            [Wr0[:, sl], z64, wk0_pad[:, sl]], axis=0))
        cols.append(jnp.concatenate(
            [s * Wk1[:, sl], s * Wr1[:, sl], z128x], axis=0))
        bias.append(s * b0[sl])
        bias.append(s * b1[sl])
    w_big = jnp.concatenate(cols, axis=1)
    bias_big = jnp.concatenate(bias).reshape(1, 8 * UNITS)
    return w_big, bias_big


def _rnn_call(xs, w_big, bias_big, Wd1, bd1, Wd2, bd2, interpret=False):
    return pl.pallas_call(
        _rnn_body,
        out_shape=jax.ShapeDtypeStruct((B, 1), jnp.float32),
        interpret=interpret,
    )(xs, w_big, bias_big, Wd1, bd1.reshape(1, UNITS), Wd2,
      bd2.reshape(1, 1))


def kernel(inputs, emb, Wk0, Wr0, b0, Wk1, Wr1, b1, Wd1, bd1, Wd2, bd2):
    emb_pad = jnp.concatenate(
        [emb, jnp.zeros((VOCAB, EMB_PAD - EMB), emb.dtype)], axis=1)
    idx = jnp.transpose(inputs).reshape(NTOK)
    xs = _make_sc_gather()(emb_pad, idx)
    w_big, bias_big = _build_weights(Wk0, Wr0, b0, Wk1, Wr1, b1)
    return _rnn_call(xs, w_big, bias_big, Wd1, bd1, Wd2, bd2)


# speedup vs baseline: 1.1395x; 1.0267x over previous
import functools

import jax
import jax.numpy as jnp
from jax import lax
from jax.experimental import pallas as pl
from jax.experimental.pallas import tpu as pltpu
from jax.experimental.pallas import tpu_sc as plsc

B = 128
S = 80
VOCAB = 10000
EMB = 100
EMB_PAD = 128
UNITS = 64
NTOK = B * S

NC = 2
NS = 16
NW = NC * NS
ROWS_PER_W = NTOK // NW


@functools.lru_cache(maxsize=1)
def _make_sc_gather():
    mesh = plsc.VectorSubcoreMesh(core_axis_name="c", subcore_axis_name="s")

    @functools.partial(
        pl.kernel,
        mesh=mesh,
        out_type=jax.ShapeDtypeStruct((NTOK, EMB_PAD), jnp.float32),
        scratch_types=[
            pltpu.VMEM((ROWS_PER_W,), jnp.int32),
            pltpu.VMEM((ROWS_PER_W, EMB_PAD), jnp.float32),
            pltpu.SemaphoreType.DMA,
        ],
    )
    def _sc_gather(table_hbm, idx_hbm, out_hbm, idx_v, rows_v, sem):
        wid = lax.axis_index("s") * NC + lax.axis_index("c")
        base = wid * ROWS_PER_W
        pltpu.sync_copy(idx_hbm.at[pl.ds(base, ROWS_PER_W)], idx_v)
        pltpu.async_copy(table_hbm.at[idx_v], rows_v, sem).wait()
        pltpu.sync_copy(rows_v, out_hbm.at[pl.ds(base, ROWS_PER_W)])

    return _sc_gather


def _rnn_body(xs_ref, w_ref, bias_ref, wd1_ref, bd1_ref, wd2_ref,
              bd2_ref, out_ref):
    H2 = 2 * UNITS

    def gates(Z):
        T = jnp.tanh(Z)
        U = 0.5 * T + 0.5
        return U[:, 0:H2], U[:, H2:2 * H2], T[:, 2 * H2:3 * H2], \
            U[:, 3 * H2:4 * H2]

    colmask = lax.broadcasted_iota(jnp.int32, (B, H2), 1) >= UNITS
    Z0 = jnp.dot(xs_ref[pl.ds(0, B), :], w_ref[pl.ds(H2, EMB_PAD), :],
                 preferred_element_type=jnp.float32) + bias_ref[...]
    i, f, g, o = gates(Z0)
    C = jnp.where(colmask, 0.0, i * g)
    H = jnp.where(colmask, 0.0, o * jnp.tanh(C))

    HB = B // 2
    Ha, Ca, Hb, Cb = H[0:HB], C[0:HB], H[HB:B], C[HB:B]
    Zb = jnp.dot(jnp.concatenate([Hb, xs_ref[pl.ds(B + HB, HB), :]], axis=1),
                 w_ref[...], preferred_element_type=jnp.float32) + bias_ref[...]

    def step(r, carry):
        Ha, Ca, Hb, Cb, Zb = carry
        ib, fb, gb, ob = gates(Zb)
        Cb = fb * Cb + ib * gb
        Hb = ob * jnp.tanh(Cb)
        tx = jnp.minimum(r, S - 1) * B
        Aa = jnp.concatenate([Ha, xs_ref[pl.ds(tx, HB), :]], axis=1)
        Za = jnp.dot(Aa, w_ref[...],
                     preferred_element_type=jnp.float32) + bias_ref[...]
        ia, fa, ga, oa = gates(Za)
        Ca = fa * Ca + ia * ga
        Ha = oa * jnp.tanh(Ca)
        tx2 = jnp.minimum(r + 1, S - 1) * B
        Ab = jnp.concatenate([Hb, xs_ref[pl.ds(tx2 + HB, HB), :]], axis=1)
        Zb = jnp.dot(Ab, w_ref[...],
                     preferred_element_type=jnp.float32) + bias_ref[...]
        return Ha, Ca, Hb, Cb, Zb

    Ha, Ca, Hb, Cb, Zb = lax.fori_loop(1, S + 1, step, (Ha, Ca, Hb, Cb, Zb),
                                       unroll=8)
    H = jnp.concatenate([Ha, Hb], axis=0)
    h1 = H[:, UNITS:H2]

    hidden = jnp.maximum(
        jnp.dot(h1, wd1_ref[...], preferred_element_type=jnp.float32)
        + bd1_ref[...], 0.0)
    logits = jnp.dot(hidden, wd2_ref[...],
                     preferred_element_type=jnp.float32) + bd2_ref[...]
    out_ref[...] = jax.nn.sigmoid(logits)


def _build_weights(Wk0, Wr0, b0, Wk1, Wr1, b1):
    wk0_pad = jnp.concatenate(
        [Wk0, jnp.zeros((EMB_PAD - EMB, 4 * UNITS), Wk0.dtype)], axis=0)
    z64 = jnp.zeros((UNITS, UNITS), jnp.float32)
    z128x = jnp.zeros((EMB_PAD, UNITS), jnp.float32)
    cols = []
    bias = []
    for gi, gate in enumerate("ifgo"):
        s = 1.0 if gate == "g" else 0.5
        sl = slice(gi * UNITS, (gi + 1) * UNITS)
        cols.append(s * jnp.concatenate(
            [Wr0[:, sl], z64, wk0_pad[:, sl]], axis=0))
        cols.append(jnp.concatenate(
            [s * Wk1[:, sl], s * Wr1[:, sl], z128x], axis=0))
        bias.append(s * b0[sl])
        bias.append(s * b1[sl])
    w_big = jnp.concatenate(cols, axis=1)
    bias_big = jnp.concatenate(bias).reshape(1, 8 * UNITS)
    return w_big, bias_big


def _rnn_call(xs, w_big, bias_big, Wd1, bd1, Wd2, bd2, interpret=False):
    return pl.pallas_call(
        _rnn_body,
        out_shape=jax.ShapeDtypeStruct((B, 1), jnp.float32),
        interpret=interpret,
    )(xs, w_big, bias_big, Wd1, bd1.reshape(1, UNITS), Wd2,
      bd2.reshape(1, 1))


def kernel(inputs, emb, Wk0, Wr0, b0, Wk1, Wr1, b1, Wd1, bd1, Wd2, bd2):
    emb_pad = jnp.concatenate(
        [emb, jnp.zeros((VOCAB, EMB_PAD - EMB), emb.dtype)], axis=1)
    idx = jnp.transpose(inputs).reshape(NTOK)
    xs = _make_sc_gather()(emb_pad, idx)
    w_big, bias_big = _build_weights(Wk0, Wr0, b0, Wk1, Wr1, b1)
    return _rnn_call(xs, w_big, bias_big, Wd1, bd1, Wd2, bd2)


# speedup vs baseline: 1.1606x; 1.0185x over previous
import functools

import jax
import jax.numpy as jnp
from jax import lax
from jax.experimental import pallas as pl
from jax.experimental.pallas import tpu as pltpu
from jax.experimental.pallas import tpu_sc as plsc

B = 128
S = 80
VOCAB = 10000
EMB = 100
EMB_PAD = 128
UNITS = 64
NTOK = B * S

NC = 2
NS = 16
NW = NC * NS
ROWS_PER_W = NTOK // NW


@functools.lru_cache(maxsize=1)
def _make_sc_gather():
    mesh = plsc.VectorSubcoreMesh(core_axis_name="c", subcore_axis_name="s")

    @functools.partial(
        pl.kernel,
        mesh=mesh,
        out_type=jax.ShapeDtypeStruct((NTOK, EMB_PAD), jnp.float32),
        scratch_types=[
            pltpu.VMEM((ROWS_PER_W,), jnp.int32),
            pltpu.VMEM((ROWS_PER_W, EMB_PAD), jnp.float32),
            pltpu.SemaphoreType.DMA,
        ],
    )
    def _sc_gather(table_hbm, idx_hbm, out_hbm, idx_v, rows_v, sem):
        wid = lax.axis_index("s") * NC + lax.axis_index("c")
        base = wid * ROWS_PER_W
        pltpu.sync_copy(idx_hbm.at[pl.ds(base, ROWS_PER_W)], idx_v)
        pltpu.async_copy(table_hbm.at[idx_v], rows_v, sem).wait()
        pltpu.sync_copy(rows_v, out_hbm.at[pl.ds(base, ROWS_PER_W)])

    return _sc_gather


def _rnn_body(xs_ref, w_ref, bias_ref, wd1_ref, bd1_ref, wd2_ref,
              bd2_ref, out_ref):
    H2 = 2 * UNITS

    def gates(Z):
        T = jnp.tanh(Z)
        U = 0.5 * T + 0.5
        return U[:, 0:H2], U[:, H2:2 * H2], T[:, 2 * H2:3 * H2], \
            U[:, 3 * H2:4 * H2]

    colmask = lax.broadcasted_iota(jnp.int32, (B, H2), 1) >= UNITS
    Z0 = jnp.dot(xs_ref[pl.ds(0, B), :], w_ref[pl.ds(H2, EMB_PAD), :],
                 preferred_element_type=jnp.float32) + bias_ref[...]
    i, f, g, o = gates(Z0)
    C = jnp.where(colmask, 0.0, i * g)
    H = jnp.where(colmask, 0.0, o * jnp.tanh(C))

    HB = B // 2
    Ha, Ca, Hb, Cb = H[0:HB], C[0:HB], H[HB:B], C[HB:B]
    Zb = jnp.dot(jnp.concatenate([Hb, xs_ref[pl.ds(B + HB, HB), :]], axis=1),
                 w_ref[...], preferred_element_type=jnp.float32) + bias_ref[...]

    def step(r, carry):
        Ha, Ca, Hb, Cb, Zb = carry
        ib, fb, gb, ob = gates(Zb)
        Cb = fb * Cb + ib * gb
        Hb = ob * jnp.tanh(Cb)
        tx = jnp.minimum(r, S - 1) * B
        Aa = jnp.concatenate([Ha, xs_ref[pl.ds(tx, HB), :]], axis=1)
        Za = jnp.dot(Aa, w_ref[...],
                     preferred_element_type=jnp.float32) + bias_ref[...]
        ia, fa, ga, oa = gates(Za)
        Ca = fa * Ca + ia * ga
        Ha = oa * jnp.tanh(Ca)
        tx2 = jnp.minimum(r + 1, S - 1) * B
        Ab = jnp.concatenate([Hb, xs_ref[pl.ds(tx2 + HB, HB), :]], axis=1)
        Zb = jnp.dot(Ab, w_ref[...],
                     preferred_element_type=jnp.float32) + bias_ref[...]
        return Ha, Ca, Hb, Cb, Zb

    Ha, Ca, Hb, Cb, Zb = lax.fori_loop(1, S + 1, step, (Ha, Ca, Hb, Cb, Zb),
                                       unroll=16)
    H = jnp.concatenate([Ha, Hb], axis=0)
    h1 = H[:, UNITS:H2]

    hidden = jnp.maximum(
        jnp.dot(h1, wd1_ref[...], preferred_element_type=jnp.float32)
        + bd1_ref[...], 0.0)
    logits = jnp.dot(hidden, wd2_ref[...],
                     preferred_element_type=jnp.float32) + bd2_ref[...]
    out_ref[...] = jax.nn.sigmoid(logits)


def _build_weights(Wk0, Wr0, b0, Wk1, Wr1, b1):
    wk0_pad = jnp.concatenate(
        [Wk0, jnp.zeros((EMB_PAD - EMB, 4 * UNITS), Wk0.dtype)], axis=0)
    z64 = jnp.zeros((UNITS, UNITS), jnp.float32)
    z128x = jnp.zeros((EMB_PAD, UNITS), jnp.float32)
    cols = []
    bias = []
    for gi, gate in enumerate("ifgo"):
        s = 1.0 if gate == "g" else 0.5
        sl = slice(gi * UNITS, (gi + 1) * UNITS)
        cols.append(s * jnp.concatenate(
            [Wr0[:, sl], z64, wk0_pad[:, sl]], axis=0))
        cols.append(jnp.concatenate(
            [s * Wk1[:, sl], s * Wr1[:, sl], z128x], axis=0))
        bias.append(s * b0[sl])
        bias.append(s * b1[sl])
    w_big = jnp.concatenate(cols, axis=1)
    bias_big = jnp.concatenate(bias).reshape(1, 8 * UNITS)
    return w_big, bias_big


def _rnn_call(xs, w_big, bias_big, Wd1, bd1, Wd2, bd2, interpret=False):
    return pl.pallas_call(
        _rnn_body,
        out_shape=jax.ShapeDtypeStruct((B, 1), jnp.float32),
        interpret=interpret,
    )(xs, w_big, bias_big, Wd1, bd1.reshape(1, UNITS), Wd2,
      bd2.reshape(1, 1))


def kernel(inputs, emb, Wk0, Wr0, b0, Wk1, Wr1, b1, Wd1, bd1, Wd2, bd2):
    emb_pad = jnp.concatenate(
        [emb, jnp.zeros((VOCAB, EMB_PAD - EMB), emb.dtype)], axis=1)
    idx = jnp.transpose(inputs).reshape(NTOK)
    xs = _make_sc_gather()(emb_pad, idx)
    w_big, bias_big = _build_weights(Wk0, Wr0, b0, Wk1, Wr1, b1)
    return _rnn_call(xs, w_big, bias_big, Wd1, bd1, Wd2, bd2)


# speedup vs baseline: 1.1697x; 1.0078x over previous
import functools

import jax
import jax.numpy as jnp
from jax import lax
from jax.experimental import pallas as pl
from jax.experimental.pallas import tpu as pltpu
from jax.experimental.pallas import tpu_sc as plsc

B = 128
S = 80
VOCAB = 10000
EMB = 100
EMB_PAD = 128
UNITS = 64
NTOK = B * S

NC = 2
NS = 16
NW = NC * NS
ROWS_PER_W = NTOK // NW


@functools.lru_cache(maxsize=1)
def _make_sc_gather():
    mesh = plsc.VectorSubcoreMesh(core_axis_name="c", subcore_axis_name="s")

    @functools.partial(
        pl.kernel,
        mesh=mesh,
        out_type=jax.ShapeDtypeStruct((NTOK, EMB_PAD), jnp.float32),
        scratch_types=[
            pltpu.VMEM((ROWS_PER_W,), jnp.int32),
            pltpu.VMEM((ROWS_PER_W, EMB_PAD), jnp.float32),
            pltpu.SemaphoreType.DMA,
        ],
    )
    def _sc_gather(table_hbm, idx_hbm, out_hbm, idx_v, rows_v, sem):
        wid = lax.axis_index("s") * NC + lax.axis_index("c")
        base = wid * ROWS_PER_W
        pltpu.sync_copy(idx_hbm.at[pl.ds(base, ROWS_PER_W)], idx_v)
        pltpu.async_copy(table_hbm.at[idx_v], rows_v, sem).wait()
        pltpu.sync_copy(rows_v, out_hbm.at[pl.ds(base, ROWS_PER_W)])

    return _sc_gather


def _rnn_body(xs_ref, w_ref, bias_ref, wd1_ref, bd1_ref, wd2_ref,
              bd2_ref, out_ref):
    H2 = 2 * UNITS

    def gates(Z):
        T = jnp.tanh(Z)
        U = 0.5 * T + 0.5
        return U[:, 0:H2], U[:, H2:2 * H2], T[:, 2 * H2:3 * H2], \
            U[:, 3 * H2:4 * H2]

    colmask = lax.broadcasted_iota(jnp.int32, (B, H2), 1) >= UNITS
    Z0 = jnp.dot(xs_ref[pl.ds(0, B), :], w_ref[pl.ds(H2, EMB_PAD), :],
                 preferred_element_type=jnp.float32) + bias_ref[...]
    i, f, g, o = gates(Z0)
    C = jnp.where(colmask, 0.0, i * g)
    H = jnp.where(colmask, 0.0, o * jnp.tanh(C))

    HB = B // 2
    Ha, Ca, Hb, Cb = H[0:HB], C[0:HB], H[HB:B], C[HB:B]
    Zb = jnp.dot(jnp.concatenate([Hb, xs_ref[pl.ds(B + HB, HB), :]], axis=1),
                 w_ref[...], preferred_element_type=jnp.float32) + bias_ref[...]

    def step(r, carry):
        Ha, Ca, Hb, Cb, Zb = carry
        ib, fb, gb, ob = gates(Zb)
        Cb = fb * Cb + ib * gb
        Hb = ob * jnp.tanh(Cb)
        tx = jnp.minimum(r, S - 1) * B
        Aa = jnp.concatenate([Ha, xs_ref[pl.ds(tx, HB), :]], axis=1)
        Za = jnp.dot(Aa, w_ref[...],
                     preferred_element_type=jnp.float32) + bias_ref[...]
        ia, fa, ga, oa = gates(Za)
        Ca = fa * Ca + ia * ga
        Ha = oa * jnp.tanh(Ca)
        tx2 = jnp.minimum(r + 1, S - 1) * B
        Ab = jnp.concatenate([Hb, xs_ref[pl.ds(tx2 + HB, HB), :]], axis=1)
        Zb = jnp.dot(Ab, w_ref[...],
                     preferred_element_type=jnp.float32) + bias_ref[...]
        return Ha, Ca, Hb, Cb, Zb

    Ha, Ca, Hb, Cb, Zb = lax.fori_loop(1, S + 1, step, (Ha, Ca, Hb, Cb, Zb),
                                       unroll=40)
    H = jnp.concatenate([Ha, Hb], axis=0)
    h1 = H[:, UNITS:H2]

    hidden = jnp.maximum(
        jnp.dot(h1, wd1_ref[...], preferred_element_type=jnp.float32)
        + bd1_ref[...], 0.0)
    logits = jnp.dot(hidden, wd2_ref[...],
                     preferred_element_type=jnp.float32) + bd2_ref[...]
    out_ref[...] = jax.nn.sigmoid(logits)


def _build_weights(Wk0, Wr0, b0, Wk1, Wr1, b1):
    wk0_pad = jnp.concatenate(
        [Wk0, jnp.zeros((EMB_PAD - EMB, 4 * UNITS), Wk0.dtype)], axis=0)
    z64 = jnp.zeros((UNITS, UNITS), jnp.float32)
    z128x = jnp.zeros((EMB_PAD, UNITS), jnp.float32)
    cols = []
    bias = []
    for gi, gate in enumerate("ifgo"):
        s = 1.0 if gate == "g" else 0.5
        sl = slice(gi * UNITS, (gi + 1) * UNITS)
        cols.append(s * jnp.concatenate(
            [Wr0[:, sl], z64, wk0_pad[:, sl]], axis=0))
        cols.append(jnp.concatenate(
            [s * Wk1[:, sl], s * Wr1[:, sl], z128x], axis=0))
        bias.append(s * b0[sl])
        bias.append(s * b1[sl])
    w_big = jnp.concatenate(cols, axis=1)
    bias_big = jnp.concatenate(bias).reshape(1, 8 * UNITS)
    return w_big, bias_big


def _rnn_call(xs, w_big, bias_big, Wd1, bd1, Wd2, bd2, interpret=False):
    return pl.pallas_call(
        _rnn_body,
        out_shape=jax.ShapeDtypeStruct((B, 1), jnp.float32),
        interpret=interpret,
    )(xs, w_big, bias_big, Wd1, bd1.reshape(1, UNITS), Wd2,
      bd2.reshape(1, 1))


def kernel(inputs, emb, Wk0, Wr0, b0, Wk1, Wr1, b1, Wd1, bd1, Wd2, bd2):
    emb_pad = jnp.concatenate(
        [emb, jnp.zeros((VOCAB, EMB_PAD - EMB), emb.dtype)], axis=1)
    idx = jnp.transpose(inputs).reshape(NTOK)
    xs = _make_sc_gather()(emb_pad, idx)
    w_big, bias_big = _build_weights(Wk0, Wr0, b0, Wk1, Wr1, b1)
    return _rnn_call(xs, w_big, bias_big, Wd1, bd1, Wd2, bd2)
